# baseline (speedup 1.0000x reference)
import functools

import jax
import jax.numpy as jnp
from jax import lax
from jax.experimental import pallas as pl
from jax.experimental.pallas import tpu as pltpu
from jax.experimental.pallas import tpu_sc as plsc

DIM = 1024
BATCH = 16384
NUM_CORES = 2
NUM_SUBCORES = 16
NUM_WORKERS = NUM_CORES * NUM_SUBCORES
B_PER_W = BATCH // NUM_WORKERS
CHUNK = 16
NUM_CHUNKS = B_PER_W // CHUNK
NBUF = 4


@jax.jit
def _gather(timestep, pe_matrix):
    mesh = plsc.VectorSubcoreMesh(
        core_axis_name="c", subcore_axis_name="s",
        num_cores=NUM_CORES, num_subcores=NUM_SUBCORES,
    )

    @functools.partial(
        pl.kernel,
        out_type=jax.ShapeDtypeStruct((BATCH, DIM), jnp.float32),
        mesh=mesh,
        scratch_types=[
            pltpu.VMEM((B_PER_W,), jnp.int32),
        ] + [pltpu.VMEM((CHUNK, DIM), jnp.float32) for _ in range(NBUF)]
          + [pltpu.SemaphoreType.DMA for _ in range(2 * NBUF)],
    )
    def body(idx_hbm, table_hbm, out_hbm, idx_v, *bufs_and_sems):
        bufs = bufs_and_sems[:NBUF]
        gsems = bufs_and_sems[NBUF:2 * NBUF]
        osems = bufs_and_sems[2 * NBUF:]
        wid = lax.axis_index("s") * NUM_CORES + lax.axis_index("c")
        base = wid * B_PER_W
        pltpu.sync_copy(idx_hbm.at[pl.ds(base, B_PER_W)], idx_v)

        def gather(c):
            idx_c = idx_v.at[pl.ds(c * CHUNK, CHUNK)]
            return pltpu.async_copy(table_hbm.at[idx_c], bufs[c % NBUF],
                                    gsems[c % NBUF])

        def put(c):
            dst = out_hbm.at[pl.ds(base + c * CHUNK, CHUNK)]
            return pltpu.async_copy(bufs[c % NBUF], dst, osems[c % NBUF])

        gathers = [None] * NUM_CHUNKS
        puts = [None] * NUM_CHUNKS
        gathers[0] = gather(0)
        gathers[1] = gather(1)
        for c in range(NUM_CHUNKS):
            gathers[c].wait()
            puts[c] = put(c)
            if c + 2 < NUM_CHUNKS:
                if c >= 2:
                    puts[c - 2].wait()
                gathers[c + 2] = gather(c + 2)
        for c in range(max(0, NUM_CHUNKS - 4), NUM_CHUNKS):
            puts[c].wait()

    return body(timestep, pe_matrix)


def kernel(timestep, pe_matrix):
    return _gather(timestep.astype(jnp.int32), pe_matrix)
